# baseline (speedup 1.0000x reference)
import dataclasses
import functools
import math

import jax
import jax.numpy as jnp
from jax import lax
from jax.experimental import pallas as pl
from jax.experimental.pallas import tpu as pltpu
from jax.experimental.pallas import tpu_sc as plsc

N = 10000
E = 320000
HC = 128
NRBF = 16
CUTOFF = 10.0

NC = 2
NS = 16
L = 16
NW = NC * NS

CHUNK = 40
RBLK = 8
NRBLK = N // RBLK

NSPLIT = 2
EHALF = E // NSPLIT
EPW = EHALF // NW
NCHUNK = EPW // CHUNK

_TE = 8000
HC2 = HC // 2


def _pack_bf16_pairs(v):
    vb = v.astype(jnp.bfloat16)
    lo = jax.lax.bitcast_convert_type(vb[:, :HC2], jnp.uint16).astype(jnp.uint32)
    hi = jax.lax.bitcast_convert_type(vb[:, HC2:], jnp.uint16).astype(jnp.uint32)
    return ((hi << 16) | lo).astype(jnp.int32)


def _h_body(x_ref, w_ref, o_ref):
    o_ref[...] = jnp.dot(x_ref[...], w_ref[...],
                         preferred_element_type=jnp.float32)


def _compute_h(x, lin1_w):
    return pl.pallas_call(
        _h_body,
        out_shape=jax.ShapeDtypeStruct((N, HC), jnp.float32),
    )(x, lin1_w)


def _filter_body(ea_ref, ew_ref, w1_ref, b1_ref, w2_ref, b2_ref, o_ref):
    t = jnp.tanh(jnp.dot(ea_ref[...].astype(jnp.bfloat16),
                         w1_ref[...].astype(jnp.bfloat16),
                         preferred_element_type=jnp.float32) + b1_ref[...])
    w = jnp.dot(t.astype(jnp.bfloat16), w2_ref[...].astype(jnp.bfloat16),
                preferred_element_type=jnp.float32) + b2_ref[...]
    ew = ew_ref[:, :1]
    c = 0.5 * (jnp.cos(ew * (math.pi / CUTOFF)) + 1.0)
    c = c * (ew < CUTOFF).astype(jnp.float32)
    o_ref[...] = _pack_bf16_pairs(w * c)


def _compute_w(edge_attr, edge_weight, fn1_w, fn1_b, fn2_w, fn2_b):
    ne = edge_attr.shape[0]
    grid = ne // _TE
    ew8 = jnp.broadcast_to(edge_weight.reshape(ne, 1), (ne, 16))
    return pl.pallas_call(
        _filter_body,
        grid=(grid,),
        in_specs=[
            pl.BlockSpec((_TE, NRBF), lambda i: (i, 0)),
            pl.BlockSpec((_TE, 16), lambda i: (i, 0)),
            pl.BlockSpec((NRBF, HC), lambda i: (0, 0)),
            pl.BlockSpec((1, HC), lambda i: (0, 0)),
            pl.BlockSpec((HC, HC), lambda i: (0, 0)),
            pl.BlockSpec((1, HC), lambda i: (0, 0)),
        ],
        out_specs=pl.BlockSpec((_TE, HC2), lambda i: (i, 0)),
        out_shape=jax.ShapeDtypeStruct((ne, HC2), jnp.int32),
    )(edge_attr, ew8, fn1_w, fn1_b.reshape(1, HC),
      fn2_w, fn2_b.reshape(1, HC))


def _sc_conv_body(h_hbm, w_hbm, src_hbm, dst_hbm, out_hbm,
                  src0, src1, dstS0, dstS1, rows0, rows1, wv0, wv1, acc_sh,
                  sem_i0, sem_i1, sem_d0, sem_d1, sem_g0, sem_g1,
                  sem_w0, sem_w1, sem_s0, sem_s1):
    cid = lax.axis_index("c")
    sid = lax.axis_index("s")
    wid = cid * NS + sid
    base = wid * EPW

    srcb = (src0, src1)
    dstb = (dstS0, dstS1)
    rowsb = (rows0, rows1)
    wb = (wv0, wv1)
    sem_i = (sem_i0, sem_i1)
    sem_d = (sem_d0, sem_d1)
    sem_g = (sem_g0, sem_g1)
    sem_w = (sem_w0, sem_w1)
    sem_s = (sem_s0, sem_s1)

    @pl.loop(0, RBLK)
    def _zero_rows(r):
        @pl.loop(0, HC, step=L)
        def _zero_lanes(k):
            rows0[r, pl.ds(k, L)] = jnp.zeros((L,), jnp.float32)

    @pl.loop(sid, NRBLK, step=NS)
    def _zero_acc(b):
        pltpu.sync_copy(rows0.at[pl.ds(0, RBLK)],
                        acc_sh.at[pl.ds(b * RBLK, RBLK)])

    plsc.subcore_barrier()

    def issue_src(c, p):
        pltpu.async_copy(src_hbm.at[pl.ds(base + c * CHUNK, CHUNK)],
                         srcb[p], sem_i[p])

    def wait_src(p):
        pltpu.make_async_copy(src_hbm.at[pl.ds(0, CHUNK)], srcb[p],
                              sem_i[p]).wait()

    def issue_dst(c, p):
        pltpu.async_copy(dst_hbm.at[pl.ds(base + c * CHUNK, CHUNK)],
                         dstb[p], sem_d[p])

    def wait_dst(p):
        pltpu.make_async_copy(dst_hbm.at[pl.ds(0, CHUNK)], dstb[p],
                              sem_d[p]).wait()

    def issue_gw(c, p):
        pltpu.async_copy(h_hbm.at[srcb[p]], rowsb[p], sem_g[p])
        pltpu.async_copy(w_hbm.at[pl.ds(base + c * CHUNK, CHUNK)],
                         wb[p], sem_w[p])

    def wait_gw(p):
        pltpu.make_async_copy(h_hbm.at[srcb[p]], rowsb[p], sem_g[p]).wait()
        pltpu.make_async_copy(w_hbm.at[pl.ds(0, CHUNK)], wb[p],
                              sem_w[p]).wait()

    def issue_scatter(p):
        pltpu.async_copy(rowsb[p], acc_sh.at[dstb[p]], sem_s[p], add=True)

    def wait_scatter(p):
        pltpu.make_async_copy(rowsb[p], acc_sh.at[dstb[p]], sem_s[p]).wait()

    def multiply(p):
        rv, wv = rowsb[p], wb[p]
        hi = jnp.full((L,), -65536, dtype=jnp.int32)

        @pl.loop(0, CHUNK)
        def _row(r):
            @pl.loop(0, HC2, step=L)
            def _lane(k):
                w32 = wv[r, pl.ds(k, L)]
                wa = plsc.bitcast(w32 << 16, jnp.float32)
                wb_ = plsc.bitcast(w32 & hi, jnp.float32)
                rv[r, pl.ds(k, L)] = rv[r, pl.ds(k, L)] * wa
                rv[r, pl.ds(k + HC2, L)] = rv[r, pl.ds(k + HC2, L)] * wb_

    def stage(c, p, first=False, issue_next=True, issue_src2=True):
        q = 1 - p
        if not first:
            wait_scatter(q)
        if issue_next:
            wait_src(q)
            issue_gw(c + 1, q)
            issue_dst(c + 1, q)
        wait_gw(p)
        if issue_src2:
            issue_src(c + 2, p)
        multiply(p)
        wait_dst(p)
        issue_scatter(p)

    issue_src(0, 0)
    issue_src(1, 1)
    issue_dst(0, 0)
    wait_src(0)
    issue_gw(0, 0)

    stage(0, 0, first=True)
    stage(1, 1)
    stage(2, 0)

    @pl.loop(0, (NCHUNK - 5) // 2)
    def _pair(t):
        c0 = 3 + 2 * t
        stage(c0, 1)
        stage(c0 + 1, 0)

    stage(NCHUNK - 2, 1, issue_src2=False)
    stage(NCHUNK - 1, 0, issue_next=False, issue_src2=False)
    wait_scatter(0)

    plsc.subcore_barrier()

    @pl.loop(sid, NRBLK, step=NS)
    def _writeback(b):
        pltpu.sync_copy(acc_sh.at[pl.ds(b * RBLK, RBLK)],
                        out_hbm.at[pl.ds(cid * N + b * RBLK, RBLK)])


def _sc_conv(h, wmat, src, dst):
    mesh = plsc.VectorSubcoreMesh(core_axis_name="c", subcore_axis_name="s",
                                  num_cores=NC, num_subcores=NS)
    cp = pltpu.CompilerParams()
    if "needs_layout_passes" in pltpu.CompilerParams.__dataclass_fields__:
        cp = dataclasses.replace(cp, needs_layout_passes=False)
    kern = pl.kernel(
        _sc_conv_body,
        out_type=jax.ShapeDtypeStruct((NC * N, HC), jnp.float32),
        mesh=mesh,
        compiler_params=cp,
        scratch_types=[
            pltpu.VMEM((CHUNK,), jnp.int32),
            pltpu.VMEM((CHUNK,), jnp.int32),
            pltpu.VMEM((CHUNK,), jnp.int32),
            pltpu.VMEM((CHUNK,), jnp.int32),
            pltpu.VMEM((CHUNK, HC), jnp.float32),
            pltpu.VMEM((CHUNK, HC), jnp.float32),
            pltpu.VMEM((CHUNK, HC2), jnp.int32),
            pltpu.VMEM((CHUNK, HC2), jnp.int32),
            pltpu.VMEM_SHARED((N, HC), jnp.float32),
        ] + [pltpu.SemaphoreType.DMA] * 10,
    )
    return kern(h, wmat, src, dst)


def _out_body(p0_ref, p1_ref, w2_ref, b2_ref, wo_ref, bo_ref, o_ref):
    agg = (p0_ref[:N, :] + p0_ref[N:, :]) + (p1_ref[:N, :] + p1_ref[N:, :])
    conv = jnp.dot(agg, w2_ref[...], preferred_element_type=jnp.float32) + b2_ref[...]
    o_ref[...] = jnp.dot(jnp.tanh(conv), wo_ref[...],
                         preferred_element_type=jnp.float32) + bo_ref[...]


def _compute_out(parts0, parts1, lin2_w, lin2_b, lin_w, lin_b):
    return pl.pallas_call(
        _out_body,
        out_shape=jax.ShapeDtypeStruct((N, HC), jnp.float32),
    )(parts0, parts1, lin2_w, lin2_b.reshape(1, HC), lin_w, lin_b.reshape(1, HC))


def kernel(x, edge_index, edge_weight, edge_attr,
           lin1_w, fn1_w, fn1_b, fn2_w, fn2_b, lin2_w, lin2_b, lin_w, lin_b):
    src = edge_index[0]
    dst = edge_index[1]
    h = _compute_h(x, lin1_w)
    parts = []
    for s in range(NSPLIT):
        lo, hi = s * EHALF, (s + 1) * EHALF
        wm = _compute_w(edge_attr[lo:hi], edge_weight[lo:hi],
                        fn1_w, fn1_b, fn2_w, fn2_b)
        parts.append(_sc_conv(h, wm, src[lo:hi], dst[lo:hi]))
    return _compute_out(parts[0], parts[1], lin2_w, lin2_b, lin_w, lin_b)


# speedup vs baseline: 1.6518x; 1.6518x over previous
import dataclasses
import functools
import math

import jax
import jax.numpy as jnp
from jax import lax
from jax.experimental import pallas as pl
from jax.experimental.pallas import tpu as pltpu
from jax.experimental.pallas import tpu_sc as plsc

N = 10000
E = 320000
HC = 128
NRBF = 16
CUTOFF = 10.0

NC = 2
NS = 16
L = 16
NW = NC * NS

CHUNK = 40
RBLK = 8
NRBLK = N // RBLK

NSPLIT = 2
EHALF = E // NSPLIT
EPW = EHALF // NW
NCHUNK = EPW // CHUNK

_TE = 6400
HC2 = HC // 2


def _pack_bf16_pairs(v):
    vb = v.astype(jnp.bfloat16)
    lo = jax.lax.bitcast_convert_type(vb[:, :HC2], jnp.uint16).astype(jnp.uint32)
    hi = jax.lax.bitcast_convert_type(vb[:, HC2:], jnp.uint16).astype(jnp.uint32)
    return ((hi << 16) | lo).astype(jnp.int32)


def _h_body(x_ref, w_ref, ew_ref, o_ref, c_ref):
    o_ref[...] = jnp.dot(x_ref[...], w_ref[...],
                         preferred_element_type=jnp.float32)
    ew = ew_ref[...]
    c2d = 0.5 * (jnp.cos(ew * (math.pi / CUTOFF)) + 1.0)
    c_ref[...] = c2d * (ew < CUTOFF).astype(jnp.float32)


def _compute_h(x, lin1_w, edge_weight):
    h, c2d = pl.pallas_call(
        _h_body,
        out_shape=(jax.ShapeDtypeStruct((N, HC), jnp.float32),
                   jax.ShapeDtypeStruct((E // 128, 128), jnp.float32)),
    )(x, lin1_w, edge_weight.reshape(E // 128, 128))
    return h, c2d.reshape(E)


def _filter_body(ea_ref, w1_ref, b1_ref, w2_ref, b2_ref, o_ref):
    t = jnp.tanh(jnp.dot(ea_ref[...].astype(jnp.bfloat16),
                         w1_ref[...].astype(jnp.bfloat16),
                         preferred_element_type=jnp.float32) + b1_ref[...])
    w = jnp.dot(t.astype(jnp.bfloat16), w2_ref[...].astype(jnp.bfloat16),
                preferred_element_type=jnp.float32) + b2_ref[...]
    o_ref[...] = _pack_bf16_pairs(w)


def _compute_w(edge_attr, fn1_w, fn1_b, fn2_w, fn2_b, half):
    grid = EHALF // _TE
    off = half * (EHALF // _TE)
    return pl.pallas_call(
        _filter_body,
        grid=(grid,),
        in_specs=[
            pl.BlockSpec((_TE, NRBF), lambda i: (i + off, 0)),
            pl.BlockSpec((NRBF, HC), lambda i: (0, 0)),
            pl.BlockSpec((1, HC), lambda i: (0, 0)),
            pl.BlockSpec((HC, HC), lambda i: (0, 0)),
            pl.BlockSpec((1, HC), lambda i: (0, 0)),
        ],
        out_specs=pl.BlockSpec((_TE, HC2), lambda i: (i, 0)),
        out_shape=jax.ShapeDtypeStruct((EHALF, HC2), jnp.int32),
    )(edge_attr, fn1_w, fn1_b.reshape(1, HC),
      fn2_w, fn2_b.reshape(1, HC))


def _sc_conv_body(h_hbm, w_hbm, src_hbm, dst_hbm, c_hbm, out_hbm,
                  src0, src1, dstS0, dstS1, rows0, rows1, wv0, wv1, cv0, cv1,
                  acc_sh,
                  sem_i0, sem_i1, sem_d0, sem_d1, sem_g0, sem_g1,
                  sem_w0, sem_w1, sem_s0, sem_s1, sem_c0, sem_c1, half):
    cid = lax.axis_index("c")
    sid = lax.axis_index("s")
    wid = cid * NS + sid
    base = half * EHALF + wid * EPW
    wbase = wid * EPW

    srcb = (src0, src1)
    dstb = (dstS0, dstS1)
    rowsb = (rows0, rows1)
    wb = (wv0, wv1)
    cb = (cv0, cv1)
    sem_c = (sem_c0, sem_c1)
    sem_i = (sem_i0, sem_i1)
    sem_d = (sem_d0, sem_d1)
    sem_g = (sem_g0, sem_g1)
    sem_w = (sem_w0, sem_w1)
    sem_s = (sem_s0, sem_s1)

    @pl.loop(0, RBLK)
    def _zero_rows(r):
        @pl.loop(0, HC, step=L)
        def _zero_lanes(k):
            rows0[r, pl.ds(k, L)] = jnp.zeros((L,), jnp.float32)

    @pl.loop(sid, NRBLK, step=NS)
    def _zero_acc(b):
        pltpu.sync_copy(rows0.at[pl.ds(0, RBLK)],
                        acc_sh.at[pl.ds(b * RBLK, RBLK)])

    plsc.subcore_barrier()

    def issue_src(c, p):
        pltpu.async_copy(src_hbm.at[pl.ds(base + c * CHUNK, CHUNK)],
                         srcb[p], sem_i[p])

    def wait_src(p):
        pltpu.make_async_copy(src_hbm.at[pl.ds(0, CHUNK)], srcb[p],
                              sem_i[p]).wait()

    def issue_dst(c, p):
        pltpu.async_copy(dst_hbm.at[pl.ds(base + c * CHUNK, CHUNK)],
                         dstb[p], sem_d[p])

    def wait_dst(p):
        pltpu.make_async_copy(dst_hbm.at[pl.ds(0, CHUNK)], dstb[p],
                              sem_d[p]).wait()

    def issue_gw(c, p):
        pltpu.async_copy(h_hbm.at[srcb[p]], rowsb[p], sem_g[p])
        pltpu.async_copy(w_hbm.at[pl.ds(wbase + c * CHUNK, CHUNK)],
                         wb[p], sem_w[p])
        pltpu.async_copy(c_hbm.at[pl.ds(base + c * CHUNK, CHUNK)],
                         cb[p], sem_c[p])

    def wait_gw(p):
        pltpu.make_async_copy(h_hbm.at[srcb[p]], rowsb[p], sem_g[p]).wait()
        pltpu.make_async_copy(w_hbm.at[pl.ds(0, CHUNK)], wb[p],
                              sem_w[p]).wait()
        pltpu.make_async_copy(c_hbm.at[pl.ds(0, CHUNK)], cb[p],
                              sem_c[p]).wait()

    def issue_scatter(p):
        pltpu.async_copy(rowsb[p], acc_sh.at[dstb[p]], sem_s[p], add=True)

    def wait_scatter(p):
        pltpu.make_async_copy(rowsb[p], acc_sh.at[dstb[p]], sem_s[p]).wait()

    def multiply(p):
        rv, wv, cv = rowsb[p], wb[p], cb[p]
        hi = jnp.full((L,), -65536, dtype=jnp.int32)

        @pl.loop(0, CHUNK)
        def _row(r):
            csp = plsc.load_gather(cv, [jnp.full((L,), r, jnp.int32)])

            @pl.loop(0, HC2, step=L)
            def _lane(k):
                w32 = wv[r, pl.ds(k, L)]
                wa = plsc.bitcast(w32 << 16, jnp.float32) * csp
                wb_ = plsc.bitcast(w32 & hi, jnp.float32) * csp
                rv[r, pl.ds(k, L)] = rv[r, pl.ds(k, L)] * wa
                rv[r, pl.ds(k + HC2, L)] = rv[r, pl.ds(k + HC2, L)] * wb_

    def stage(c, p, first=False, issue_next=True, issue_src2=True):
        q = 1 - p
        if not first:
            wait_scatter(q)
        if issue_next:
            wait_src(q)
            issue_gw(c + 1, q)
            issue_dst(c + 1, q)
        wait_gw(p)
        if issue_src2:
            issue_src(c + 2, p)
        multiply(p)
        wait_dst(p)
        issue_scatter(p)

    issue_src(0, 0)
    issue_src(1, 1)
    issue_dst(0, 0)
    wait_src(0)
    issue_gw(0, 0)

    stage(0, 0, first=True)
    stage(1, 1)
    stage(2, 0)

    @pl.loop(0, (NCHUNK - 5) // 2)
    def _pair(t):
        c0 = 3 + 2 * t
        stage(c0, 1)
        stage(c0 + 1, 0)

    stage(NCHUNK - 2, 1, issue_src2=False)
    stage(NCHUNK - 1, 0, issue_next=False, issue_src2=False)
    wait_scatter(0)

    plsc.subcore_barrier()

    @pl.loop(sid, NRBLK, step=NS)
    def _writeback(b):
        pltpu.sync_copy(acc_sh.at[pl.ds(b * RBLK, RBLK)],
                        out_hbm.at[pl.ds(cid * N + b * RBLK, RBLK)])


def _sc_conv(h, wmat, src, dst, cvec, half):
    mesh = plsc.VectorSubcoreMesh(core_axis_name="c", subcore_axis_name="s",
                                  num_cores=NC, num_subcores=NS)
    cp = pltpu.CompilerParams()
    if "needs_layout_passes" in pltpu.CompilerParams.__dataclass_fields__:
        cp = dataclasses.replace(cp, needs_layout_passes=False)
    kern = pl.kernel(
        functools.partial(_sc_conv_body, half=half),
        out_type=jax.ShapeDtypeStruct((NC * N, HC), jnp.float32),
        mesh=mesh,
        compiler_params=cp,
        scratch_types=[
            pltpu.VMEM((CHUNK,), jnp.int32),
            pltpu.VMEM((CHUNK,), jnp.int32),
            pltpu.VMEM((CHUNK,), jnp.int32),
            pltpu.VMEM((CHUNK,), jnp.int32),
            pltpu.VMEM((CHUNK, HC), jnp.float32),
            pltpu.VMEM((CHUNK, HC), jnp.float32),
            pltpu.VMEM((CHUNK, HC2), jnp.int32),
            pltpu.VMEM((CHUNK, HC2), jnp.int32),
            pltpu.VMEM((CHUNK,), jnp.float32),
            pltpu.VMEM((CHUNK,), jnp.float32),
            pltpu.VMEM_SHARED((N, HC), jnp.float32),
        ] + [pltpu.SemaphoreType.DMA] * 12,
    )
    return kern(h, wmat, src, dst, cvec)


def _out_body(p0_ref, p1_ref, w2_ref, b2_ref, wo_ref, bo_ref, o_ref):
    agg = (p0_ref[:N, :] + p0_ref[N:, :]) + (p1_ref[:N, :] + p1_ref[N:, :])
    conv = jnp.dot(agg, w2_ref[...], preferred_element_type=jnp.float32) + b2_ref[...]
    o_ref[...] = jnp.dot(jnp.tanh(conv), wo_ref[...],
                         preferred_element_type=jnp.float32) + bo_ref[...]


def _compute_out(parts0, parts1, lin2_w, lin2_b, lin_w, lin_b):
    return pl.pallas_call(
        _out_body,
        out_shape=jax.ShapeDtypeStruct((N, HC), jnp.float32),
    )(parts0, parts1, lin2_w, lin2_b.reshape(1, HC), lin_w, lin_b.reshape(1, HC))


def kernel(x, edge_index, edge_weight, edge_attr,
           lin1_w, fn1_w, fn1_b, fn2_w, fn2_b, lin2_w, lin2_b, lin_w, lin_b):
    src = edge_index[0]
    dst = edge_index[1]
    h, cvec = _compute_h(x, lin1_w, edge_weight)
    parts = []
    for s in range(NSPLIT):
        wm = _compute_w(edge_attr, fn1_w, fn1_b, fn2_w, fn2_b, s)
        parts.append(_sc_conv(h, wm, src, dst, cvec, s))
    return _compute_out(parts[0], parts[1], lin2_w, lin2_b, lin_w, lin_b)


# speedup vs baseline: 1.9119x; 1.1575x over previous
import dataclasses
import functools
import math

import jax
import jax.numpy as jnp
from jax import lax
from jax.experimental import pallas as pl
from jax.experimental.pallas import tpu as pltpu
from jax.experimental.pallas import tpu_sc as plsc

N = 10000
E = 320000
HC = 128
NRBF = 16
CUTOFF = 10.0

NC = 2
NS = 16
L = 16
NW = NC * NS

CHUNK = 40
RBLK = 8
NRBLK = N // RBLK

NSPLIT = 2
EHALF = E // NSPLIT
EPW = EHALF // NW
NCHUNK = EPW // CHUNK

_TE = 8000
HC2 = HC // 2

_ZROWS = 48


def _pack_bf16_pairs(v):
    vb = v.astype(jnp.bfloat16)
    lo = jax.lax.bitcast_convert_type(vb[:, :HC2], jnp.uint16).astype(jnp.uint32)
    hi = jax.lax.bitcast_convert_type(vb[:, HC2:], jnp.uint16).astype(jnp.uint32)
    return ((hi << 16) | lo).astype(jnp.int32)


def _h_body(x_ref, w_ref, ew_ref, o_ref, c_ref):
    o_ref[...] = jnp.dot(x_ref[...], w_ref[...],
                         preferred_element_type=jnp.float32)
    ew = ew_ref[...]
    c2d = 0.5 * (jnp.cos(ew * (math.pi / CUTOFF)) + 1.0)
    c_ref[...] = c2d * (ew < CUTOFF).astype(jnp.float32)


def _compute_h(x, lin1_w, edge_weight):
    h, c2d = pl.pallas_call(
        _h_body,
        out_shape=(jax.ShapeDtypeStruct((N, HC), jnp.float32),
                   jax.ShapeDtypeStruct((E // 128, 128), jnp.float32)),
    )(x, lin1_w, edge_weight.reshape(E // 128, 128))
    return h, c2d.reshape(E)


def _filter_body(ea_ref, w1_ref, b1_ref, w2_ref, b2_ref, o_ref):
    t = jnp.tanh(jnp.dot(ea_ref[...].astype(jnp.bfloat16),
                         w1_ref[...].astype(jnp.bfloat16),
                         preferred_element_type=jnp.float32) + b1_ref[...])
    w = jnp.dot(t.astype(jnp.bfloat16), w2_ref[...].astype(jnp.bfloat16),
                preferred_element_type=jnp.float32) + b2_ref[...]
    o_ref[...] = _pack_bf16_pairs(w)


def _compute_w(edge_attr, fn1_w, fn1_b, fn2_w, fn2_b, half):
    grid = EHALF // _TE
    off = half * (EHALF // _TE)
    return pl.pallas_call(
        _filter_body,
        grid=(grid,),
        in_specs=[
            pl.BlockSpec((_TE, NRBF), lambda i: (i + off, 0)),
            pl.BlockSpec((NRBF, HC), lambda i: (0, 0)),
            pl.BlockSpec((1, HC), lambda i: (0, 0)),
            pl.BlockSpec((HC, HC), lambda i: (0, 0)),
            pl.BlockSpec((1, HC), lambda i: (0, 0)),
        ],
        out_specs=pl.BlockSpec((_TE, HC2), lambda i: (i, 0)),
        out_shape=jax.ShapeDtypeStruct((EHALF, HC2), jnp.int32),
    )(edge_attr, fn1_w, fn1_b.reshape(1, HC),
      fn2_w, fn2_b.reshape(1, HC))


def _sc_conv_body(h_hbm, w_hbm, src_hbm, dst_hbm, c_hbm, out_hbm,
                  src0, src1, dstS0, dstS1, rows0, rows1, wv0, wv1, cv0, cv1,
                  zbuf, acc_sh,
                  sem_i0, sem_i1, sem_d0, sem_d1, sem_g0, sem_g1,
                  sem_w0, sem_w1, sem_s0, sem_s1, sem_c0, sem_c1, sem_z,
                  half):
    cid = lax.axis_index("c")
    sid = lax.axis_index("s")
    wid = cid * NS + sid
    base = half * EHALF + wid * EPW
    wbase = wid * EPW

    srcb = (src0, src1)
    dstb = (dstS0, dstS1)
    rowsb = (rows0, rows1)
    wb = (wv0, wv1)
    cb = (cv0, cv1)
    sem_c = (sem_c0, sem_c1)
    sem_i = (sem_i0, sem_i1)
    sem_d = (sem_d0, sem_d1)
    sem_g = (sem_g0, sem_g1)
    sem_w = (sem_w0, sem_w1)
    sem_s = (sem_s0, sem_s1)

    @pl.loop(0, _ZROWS)
    def _zero_rows(r):
        @pl.loop(0, HC, step=L)
        def _zero_lanes(k):
            zbuf[r, pl.ds(k, L)] = jnp.zeros((L,), jnp.float32)

    zlo = sid * 624

    @pl.loop(0, 13)
    def _zero_issue(k):
        pltpu.async_copy(zbuf.at[pl.ds(0, _ZROWS)],
                         acc_sh.at[pl.ds(zlo + k * _ZROWS, _ZROWS)], sem_z)

    @pl.loop(0, 13)
    def _zero_drain(k):
        pltpu.make_async_copy(zbuf.at[pl.ds(0, _ZROWS)],
                              acc_sh.at[pl.ds(0, _ZROWS)], sem_z).wait()

    @pl.when(sid == NS - 1)
    def _zero_tail():
        pltpu.sync_copy(zbuf.at[pl.ds(0, 16)], acc_sh.at[pl.ds(9984, 16)])

    plsc.subcore_barrier()

    def issue_src(c, p):
        pltpu.async_copy(src_hbm.at[pl.ds(base + c * CHUNK, CHUNK)],
                         srcb[p], sem_i[p])

    def wait_src(p):
        pltpu.make_async_copy(src_hbm.at[pl.ds(0, CHUNK)], srcb[p],
                              sem_i[p]).wait()

    def issue_dst(c, p):
        pltpu.async_copy(dst_hbm.at[pl.ds(base + c * CHUNK, CHUNK)],
                         dstb[p], sem_d[p])

    def wait_dst(p):
        pltpu.make_async_copy(dst_hbm.at[pl.ds(0, CHUNK)], dstb[p],
                              sem_d[p]).wait()

    def issue_gw(c, p):
        pltpu.async_copy(h_hbm.at[srcb[p]], rowsb[p], sem_g[p])
        pltpu.async_copy(w_hbm.at[pl.ds(wbase + c * CHUNK, CHUNK)],
                         wb[p], sem_w[p])
        pltpu.async_copy(c_hbm.at[pl.ds(base + c * CHUNK, CHUNK)],
                         cb[p], sem_c[p])

    def wait_gw(p):
        pltpu.make_async_copy(h_hbm.at[srcb[p]], rowsb[p], sem_g[p]).wait()
        pltpu.make_async_copy(w_hbm.at[pl.ds(0, CHUNK)], wb[p],
                              sem_w[p]).wait()
        pltpu.make_async_copy(c_hbm.at[pl.ds(0, CHUNK)], cb[p],
                              sem_c[p]).wait()

    def issue_scatter(p):
        pltpu.async_copy(rowsb[p], acc_sh.at[dstb[p]], sem_s[p], add=True)

    def wait_scatter(p):
        pltpu.make_async_copy(rowsb[p], acc_sh.at[dstb[p]], sem_s[p]).wait()

    def multiply(p):
        rv, wv, cv = rowsb[p], wb[p], cb[p]
        hi = jnp.full((L,), -65536, dtype=jnp.int32)

        @pl.loop(0, CHUNK)
        def _row(r):
            csp = plsc.load_gather(cv, [jnp.full((L,), r, jnp.int32)])

            @pl.loop(0, HC2, step=L)
            def _lane(k):
                w32 = wv[r, pl.ds(k, L)]
                wa = plsc.bitcast(w32 << 16, jnp.float32) * csp
                wb_ = plsc.bitcast(w32 & hi, jnp.float32) * csp
                rv[r, pl.ds(k, L)] = rv[r, pl.ds(k, L)] * wa
                rv[r, pl.ds(k + HC2, L)] = rv[r, pl.ds(k + HC2, L)] * wb_

    def stage(c, p, first=False, issue_next=True, issue_src2=True):
        q = 1 - p
        if not first:
            wait_scatter(q)
        if issue_next:
            wait_src(q)
            issue_gw(c + 1, q)
            issue_dst(c + 1, q)
        wait_gw(p)
        if issue_src2:
            issue_src(c + 2, p)
        multiply(p)
        wait_dst(p)
        issue_scatter(p)

    issue_src(0, 0)
    issue_src(1, 1)
    issue_dst(0, 0)
    wait_src(0)
    issue_gw(0, 0)

    stage(0, 0, first=True)
    stage(1, 1)
    stage(2, 0)

    @pl.loop(0, (NCHUNK - 5) // 2)
    def _pair(t):
        c0 = 3 + 2 * t
        stage(c0, 1)
        stage(c0 + 1, 0)

    stage(NCHUNK - 2, 1, issue_src2=False)
    stage(NCHUNK - 1, 0, issue_next=False, issue_src2=False)
    wait_scatter(0)

    plsc.subcore_barrier()

    @pl.when(sid < NS - 1)
    def _wb_main():
        pltpu.sync_copy(acc_sh.at[pl.ds(zlo, 624)],
                        out_hbm.at[pl.ds(cid * N + zlo, 624)])

    @pl.when(sid == NS - 1)
    def _wb_last():
        pltpu.sync_copy(acc_sh.at[pl.ds(zlo, 640)],
                        out_hbm.at[pl.ds(cid * N + zlo, 640)])


def _sc_conv(h, wmat, src, dst, cvec, half):
    mesh = plsc.VectorSubcoreMesh(core_axis_name="c", subcore_axis_name="s",
                                  num_cores=NC, num_subcores=NS)
    cp = pltpu.CompilerParams()
    if "needs_layout_passes" in pltpu.CompilerParams.__dataclass_fields__:
        cp = dataclasses.replace(cp, needs_layout_passes=False)
    kern = pl.kernel(
        functools.partial(_sc_conv_body, half=half),
        out_type=jax.ShapeDtypeStruct((NC * N, HC), jnp.float32),
        mesh=mesh,
        compiler_params=cp,
        scratch_types=[
            pltpu.VMEM((CHUNK,), jnp.int32),
            pltpu.VMEM((CHUNK,), jnp.int32),
            pltpu.VMEM((CHUNK,), jnp.int32),
            pltpu.VMEM((CHUNK,), jnp.int32),
            pltpu.VMEM((CHUNK, HC), jnp.float32),
            pltpu.VMEM((CHUNK, HC), jnp.float32),
            pltpu.VMEM((CHUNK, HC2), jnp.int32),
            pltpu.VMEM((CHUNK, HC2), jnp.int32),
            pltpu.VMEM((CHUNK,), jnp.float32),
            pltpu.VMEM((CHUNK,), jnp.float32),
            pltpu.VMEM((_ZROWS, HC), jnp.float32),
            pltpu.VMEM_SHARED((N, HC), jnp.float32),
        ] + [pltpu.SemaphoreType.DMA] * 13,
    )
    return kern(h, wmat, src, dst, cvec)


def _out_body(p0_ref, p1_ref, w2_ref, b2_ref, wo_ref, bo_ref, o_ref):
    agg = (p0_ref[:N, :] + p0_ref[N:, :]) + (p1_ref[:N, :] + p1_ref[N:, :])
    conv = jnp.dot(agg, w2_ref[...], preferred_element_type=jnp.float32) + b2_ref[...]
    o_ref[...] = jnp.dot(jnp.tanh(conv), wo_ref[...],
                         preferred_element_type=jnp.float32) + bo_ref[...]


def _compute_out(parts0, parts1, lin2_w, lin2_b, lin_w, lin_b):
    return pl.pallas_call(
        _out_body,
        out_shape=jax.ShapeDtypeStruct((N, HC), jnp.float32),
    )(parts0, parts1, lin2_w, lin2_b.reshape(1, HC), lin_w, lin_b.reshape(1, HC))


def kernel(x, edge_index, edge_weight, edge_attr,
           lin1_w, fn1_w, fn1_b, fn2_w, fn2_b, lin2_w, lin2_b, lin_w, lin_b):
    src = edge_index[0]
    dst = edge_index[1]
    h, cvec = _compute_h(x, lin1_w, edge_weight)
    parts = []
    for s in range(NSPLIT):
        wm = _compute_w(edge_attr, fn1_w, fn1_b, fn2_w, fn2_b, s)
        parts.append(_sc_conv(h, wm, src, dst, cvec, s))
    return _compute_out(parts[0], parts[1], lin2_w, lin2_b, lin_w, lin_b)
